# baseline (speedup 1.0000x reference)
import functools

import jax
import jax.numpy as jnp
from jax import lax
from jax.experimental import pallas as pl
from jax.experimental.pallas import tpu as pltpu
from jax.experimental.pallas import tpu_sc as plsc

N_NODES = 10000
IN_DIM = 256
HID_DIM = 512
OUT_DIM = 128
N_EDGES = 160000

NP = 10112
EP = 163840
NCHUNK = EP // 128
STRIPE = NP // 16
HALF = IN_DIM // 2

_MESH = plsc.VectorSubcoreMesh(core_axis_name="c", subcore_axis_name="s")


@functools.partial(
    pl.kernel,
    out_type=jax.ShapeDtypeStruct((2, NP, 128), jnp.float32),
    mesh=_MESH,
    scratch_types=[
        pltpu.VMEM_SHARED((NP, 128), jnp.float32),
        pltpu.VMEM((40, 128), jnp.int32),
        pltpu.VMEM((128, 128), jnp.float32),
        pltpu.SemaphoreType.DMA,
    ],
)
def _sc_histogram(dst2d, zeros128, ones128, hist_out, deg_sh, dstbuf, ones_v, hsem):
    c = lax.axis_index("c")
    s = lax.axis_index("s")
    r0 = s * STRIPE
    pltpu.sync_copy(ones128, ones_v)
    pltpu.sync_copy(zeros128.at[pl.ds(r0, STRIPE)], deg_sh.at[pl.ds(r0, STRIPE)])
    base = c * (NCHUNK // 2) + s * (NCHUNK // 32)
    pltpu.sync_copy(dst2d.at[pl.ds(base, NCHUNK // 32)], dstbuf)
    plsc.subcore_barrier()

    def body(j, carry):
        pltpu.async_copy(ones_v, deg_sh.at[dstbuf.at[j]], hsem, add=True)
        return carry

    lax.fori_loop(0, NCHUNK // 32, body, 0)

    def drain(j, carry):
        pltpu.make_async_copy(ones_v, deg_sh.at[dstbuf.at[0]], hsem).wait()
        return carry

    lax.fori_loop(0, NCHUNK // 32, drain, 0)
    plsc.subcore_barrier()
    pltpu.sync_copy(deg_sh.at[pl.ds(r0, STRIPE)], hist_out.at[c, pl.ds(r0, STRIPE)])


def _tc_norm_body(hist_ref, x_ref, xs_ref, dinv_ref):
    hp = hist_ref[...]
    degsum = jnp.sum(hp[0] + hp[1], axis=1, keepdims=True)
    dinv_full = lax.rsqrt(degsum * (1.0 / 128.0) + 1.0)
    dinv = dinv_full[:N_NODES]
    xv = x_ref[...]
    xs_ref[0] = xv[:, :HALF] * dinv
    xs_ref[1] = xv[:, HALF:] * dinv
    dinv_ref[...] = dinv


def _tc_norm(hist, x):
    return pl.pallas_call(
        _tc_norm_body,
        out_shape=[
            jax.ShapeDtypeStruct((2, N_NODES, HALF), jnp.float32),
            jax.ShapeDtypeStruct((N_NODES, 1), jnp.float32),
        ],
    )(hist, x)


_CPT = NCHUNK // 16


_GCH = 16
_NG = _CPT // _GCH
_NBUF = 2


@functools.partial(
    pl.kernel,
    out_type=jax.ShapeDtypeStruct((2, NP, HALF), jnp.float32),
    mesh=_MESH,
    scratch_types=[
        pltpu.VMEM_SHARED((NP, HALF), jnp.float32),
        pltpu.VMEM((2, _GCH, 128), jnp.int32),
        pltpu.VMEM((2, _GCH, 128), jnp.int32),
        pltpu.VMEM((_NBUF, 128, HALF), jnp.float32),
        pltpu.SemaphoreType.DMA,
        pltpu.SemaphoreType.DMA,
        pltpu.SemaphoreType.DMA,
        pltpu.SemaphoreType.DMA,
        pltpu.SemaphoreType.DMA,
        pltpu.SemaphoreType.DMA,
    ],
)
def _sc_segsum(src_all, dst2d, xs_cat, zeros128, acc_out,
               acc_sh, srcbuf, dstbuf, rows,
               gs0, gs1, ss0, ss1, semis, semid):
    gsem = (gs0, gs1)
    ssem = (ss0, ss1)
    c = lax.axis_index("c")
    s = lax.axis_index("s")
    r0 = s * STRIPE
    cbase = s * _CPT
    pltpu.sync_copy(zeros128.at[pl.ds(r0, STRIPE)], acc_sh.at[pl.ds(r0, STRIPE)])
    pltpu.sync_copy(src_all.at[c, pl.ds(cbase, _GCH)], srcbuf.at[0])
    pltpu.sync_copy(dst2d.at[pl.ds(cbase, _GCH)], dstbuf.at[0])
    plsc.subcore_barrier()

    pltpu.async_copy(xs_cat.at[srcbuf.at[0, 0]], rows.at[0], gsem[0])

    def pair(p, carry):
        for b in range(_NBUF):
            j = 2 * p + b
            g = j // _GCH
            l = j - g * _GCH
            bi = jnp.remainder(g, 2)
            if b == 0:
                @pl.when(jnp.logical_and(l == 2, g < _NG - 1))
                def _():
                    off = cbase + (g + 1) * _GCH
                    nbi = jnp.remainder(g + 1, 2)
                    pltpu.async_copy(
                        src_all.at[c, pl.ds(off, _GCH)], srcbuf.at[nbi], semis)
                    pltpu.async_copy(
                        dst2d.at[pl.ds(off, _GCH)], dstbuf.at[nbi], semid)
            pltpu.make_async_copy(
                xs_cat.at[srcbuf.at[bi, l]], rows.at[b], gsem[b]).wait()
            pltpu.async_copy(
                rows.at[b], acc_sh.at[dstbuf.at[bi, l]], ssem[b], add=True)
            jf = j + 1
            bf = 1 - b
            gf = jf // _GCH
            lf = jf - gf * _GCH
            bif = jnp.remainder(gf, 2)

            @pl.when(jf < _CPT)
            def _():
                @pl.when(jf >= _NBUF)
                def _():
                    pltpu.make_async_copy(
                        rows.at[bf], acc_sh.at[dstbuf.at[0, 0]], ssem[bf]).wait()

                @pl.when(jnp.logical_and(lf == 0, gf >= 1))
                def _():
                    off = cbase + gf * _GCH
                    pltpu.make_async_copy(
                        src_all.at[c, pl.ds(off, _GCH)], srcbuf.at[bif], semis).wait()
                    pltpu.make_async_copy(
                        dst2d.at[pl.ds(off, _GCH)], dstbuf.at[bif], semid).wait()

                pltpu.async_copy(
                    xs_cat.at[srcbuf.at[bif, lf]], rows.at[bf], gsem[bf])

        return carry

    lax.fori_loop(0, _CPT // 2, pair, 0)
    pltpu.make_async_copy(rows.at[0], acc_sh.at[dstbuf.at[0, 0]], ssem[0]).wait()
    pltpu.make_async_copy(rows.at[1], acc_sh.at[dstbuf.at[0, 0]], ssem[1]).wait()
    plsc.subcore_barrier()
    pltpu.sync_copy(acc_sh.at[pl.ds(r0, STRIPE)], acc_out.at[c, pl.ds(r0, STRIPE)])


_RB = 1000


def _tc_head_body(acc_ref, xs_ref, dinv_ref, wc_ref, bc_ref, wl_ref, bl_ref, o_ref):
    acc = acc_ref[...]
    xsp = xs_ref[...]
    dinv = dinv_ref[...]
    agg = jnp.concatenate([acc[0] + xsp[0], acc[1] + xsp[1]], axis=1) * dinv
    h = jnp.dot(agg, wc_ref[...], preferred_element_type=jnp.float32) + bc_ref[...]
    h = jnp.maximum(h, 0.0)
    o_ref[...] = jnp.dot(h, wl_ref[...], preferred_element_type=jnp.float32) + bl_ref[...]


def _tc_head(acc, xs_parts, dinv, W_conv, b_conv, W_lin, b_lin):
    nblk = N_NODES // _RB
    return pl.pallas_call(
        _tc_head_body,
        grid=(nblk,),
        in_specs=[
            pl.BlockSpec((2, _RB, HALF), lambda i: (0, i, 0)),
            pl.BlockSpec((2, _RB, HALF), lambda i: (0, i, 0)),
            pl.BlockSpec((_RB, 1), lambda i: (i, 0)),
            pl.BlockSpec((IN_DIM, HID_DIM), lambda i: (0, 0)),
            pl.BlockSpec((1, HID_DIM), lambda i: (0, 0)),
            pl.BlockSpec((HID_DIM, OUT_DIM), lambda i: (0, 0)),
            pl.BlockSpec((1, OUT_DIM), lambda i: (0, 0)),
        ],
        out_specs=pl.BlockSpec((_RB, OUT_DIM), lambda i: (i, 0)),
        out_shape=jax.ShapeDtypeStruct((N_NODES, OUT_DIM), jnp.float32),
    )(acc, xs_parts, dinv, W_conv, b_conv, W_lin, b_lin)


def kernel(x, edge_index, W_conv, b_conv, W_lin, b_lin):
    src = edge_index[0].astype(jnp.int32)
    dst = edge_index[1].astype(jnp.int32)
    pad = EP - N_EDGES
    srcp = jnp.concatenate([src, jnp.zeros((pad,), jnp.int32)])
    dstp = jnp.concatenate([dst, jnp.full((pad,), N_NODES, jnp.int32)])
    dst2d = dstp.reshape(NCHUNK, 128)
    src_all = jnp.stack([srcp, srcp + N_NODES]).reshape(2, NCHUNK, 128)

    ones128 = jnp.ones((128, 128), jnp.float32)
    zeros128 = jnp.zeros((NP, HALF), jnp.float32)

    hist = _sc_histogram(dst2d, zeros128, ones128)
    xs_parts, dinv = _tc_norm(hist, x)
    xs_cat = xs_parts.reshape(2 * N_NODES, HALF)
    acc = _sc_segsum(src_all, dst2d, xs_cat, zeros128)
    return _tc_head(acc, xs_parts, dinv, W_conv,
                    b_conv.reshape(1, HID_DIM), W_lin, b_lin.reshape(1, OUT_DIM))


# speedup vs baseline: 1.0014x; 1.0014x over previous
import functools

import jax
import jax.numpy as jnp
from jax import lax
from jax.experimental import pallas as pl
from jax.experimental.pallas import tpu as pltpu
from jax.experimental.pallas import tpu_sc as plsc

N_NODES = 10000
IN_DIM = 256
HID_DIM = 512
OUT_DIM = 128
N_EDGES = 160000

NP = 10112
EP = 163840
NCHUNK = EP // 128
STRIPE = NP // 16
HALF = IN_DIM // 2

_MESH = plsc.VectorSubcoreMesh(core_axis_name="c", subcore_axis_name="s")


@functools.partial(
    pl.kernel,
    out_type=jax.ShapeDtypeStruct((2, NP, 128), jnp.float32),
    mesh=_MESH,
    scratch_types=[
        pltpu.VMEM_SHARED((NP, 128), jnp.float32),
        pltpu.VMEM((40, 128), jnp.int32),
        pltpu.VMEM((128, 128), jnp.float32),
        pltpu.SemaphoreType.DMA,
    ],
)
def _sc_histogram(dst2d, zeros128, ones128, hist_out, deg_sh, dstbuf, ones_v, hsem):
    c = lax.axis_index("c")
    s = lax.axis_index("s")
    r0 = s * STRIPE
    pltpu.sync_copy(ones128, ones_v)
    pltpu.sync_copy(zeros128.at[pl.ds(r0, STRIPE)], deg_sh.at[pl.ds(r0, STRIPE)])
    base = c * (NCHUNK // 2) + s * (NCHUNK // 32)
    pltpu.sync_copy(dst2d.at[pl.ds(base, NCHUNK // 32)], dstbuf)
    plsc.subcore_barrier()

    def body(j, carry):
        pltpu.async_copy(ones_v, deg_sh.at[dstbuf.at[j]], hsem, add=True)
        return carry

    lax.fori_loop(0, NCHUNK // 32, body, 0)

    def drain(j, carry):
        pltpu.make_async_copy(ones_v, deg_sh.at[dstbuf.at[0]], hsem).wait()
        return carry

    lax.fori_loop(0, NCHUNK // 32, drain, 0)
    plsc.subcore_barrier()
    pltpu.sync_copy(deg_sh.at[pl.ds(r0, STRIPE)], hist_out.at[c, pl.ds(r0, STRIPE)])


def _tc_norm_body(hist_ref, x_ref, xs_ref, dinv_ref):
    hp = hist_ref[...]
    degsum = jnp.sum(hp[0] + hp[1], axis=1, keepdims=True)
    dinv_full = lax.rsqrt(degsum * (1.0 / 128.0) + 1.0)
    dinv = dinv_full[:N_NODES]
    xv = x_ref[...]
    xs_ref[0] = xv[:, :HALF] * dinv
    xs_ref[1] = xv[:, HALF:] * dinv
    dinv_ref[...] = dinv


def _tc_norm(hist, x):
    return pl.pallas_call(
        _tc_norm_body,
        out_shape=[
            jax.ShapeDtypeStruct((2, N_NODES, HALF), jnp.float32),
            jax.ShapeDtypeStruct((N_NODES, 1), jnp.float32),
        ],
    )(hist, x)


_CPT = NCHUNK // 16


_GCH = 16
_NG = _CPT // _GCH
_NBUF = 2


@functools.partial(
    pl.kernel,
    out_type=jax.ShapeDtypeStruct((2, NP, HALF), jnp.float32),
    mesh=_MESH,
    scratch_types=[
        pltpu.VMEM_SHARED((NP, HALF), jnp.float32),
        pltpu.VMEM((2, _GCH, 128), jnp.int32),
        pltpu.VMEM((2, _GCH, 128), jnp.int32),
        pltpu.VMEM((_NBUF, 128, HALF), jnp.float32),
        pltpu.SemaphoreType.DMA,
        pltpu.SemaphoreType.DMA,
        pltpu.SemaphoreType.DMA,
        pltpu.SemaphoreType.DMA,
        pltpu.SemaphoreType.DMA,
        pltpu.SemaphoreType.DMA,
    ],
)
def _sc_segsum(src_all, dst2d, xs_cat, zeros128, acc_out,
               acc_sh, srcbuf, dstbuf, rows,
               gs0, gs1, ss0, ss1, semis, semid):
    gsem = (gs0, gs1)
    ssem = (ss0, ss1)
    c = lax.axis_index("c")
    s = lax.axis_index("s")
    r0 = s * STRIPE
    cbase = s * _CPT
    pltpu.sync_copy(zeros128.at[pl.ds(r0, STRIPE)], acc_sh.at[pl.ds(r0, STRIPE)])
    pltpu.sync_copy(src_all.at[c, pl.ds(cbase, _GCH)], srcbuf.at[0])
    pltpu.sync_copy(dst2d.at[pl.ds(cbase, _GCH)], dstbuf.at[0])
    plsc.subcore_barrier()

    pltpu.async_copy(xs_cat.at[srcbuf.at[0, 0]], rows.at[0], gsem[0])

    def pair(p, carry):
        for b in range(_NBUF):
            j = 2 * p + b
            g = j // _GCH
            l = j - g * _GCH
            bi = jnp.remainder(g, 2)
            if b == 0:
                @pl.when(jnp.logical_and(l == 2, g < _NG - 1))
                def _():
                    off = cbase + (g + 1) * _GCH
                    nbi = jnp.remainder(g + 1, 2)
                    pltpu.async_copy(
                        src_all.at[c, pl.ds(off, _GCH)], srcbuf.at[nbi], semis)
                    pltpu.async_copy(
                        dst2d.at[pl.ds(off, _GCH)], dstbuf.at[nbi], semid)
            pltpu.make_async_copy(
                xs_cat.at[srcbuf.at[bi, l]], rows.at[b], gsem[b]).wait()
            pltpu.async_copy(
                rows.at[b], acc_sh.at[dstbuf.at[bi, l]], ssem[b], add=True)
            jf = j + 1
            bf = 1 - b
            gf = jf // _GCH
            lf = jf - gf * _GCH
            bif = jnp.remainder(gf, 2)

            @pl.when(jf < _CPT)
            def _():
                @pl.when(jf >= _NBUF)
                def _():
                    pltpu.make_async_copy(
                        rows.at[bf], acc_sh.at[dstbuf.at[0, 0]], ssem[bf]).wait()

                @pl.when(jnp.logical_and(lf == 0, gf >= 1))
                def _():
                    off = cbase + gf * _GCH
                    pltpu.make_async_copy(
                        src_all.at[c, pl.ds(off, _GCH)], srcbuf.at[bif], semis).wait()
                    pltpu.make_async_copy(
                        dst2d.at[pl.ds(off, _GCH)], dstbuf.at[bif], semid).wait()

                pltpu.async_copy(
                    xs_cat.at[srcbuf.at[bif, lf]], rows.at[bf], gsem[bf])

        return carry

    lax.fori_loop(0, _CPT // 2, pair, 0)
    pltpu.make_async_copy(rows.at[0], acc_sh.at[dstbuf.at[0, 0]], ssem[0]).wait()
    pltpu.make_async_copy(rows.at[1], acc_sh.at[dstbuf.at[0, 0]], ssem[1]).wait()
    plsc.subcore_barrier()
    pltpu.sync_copy(acc_sh.at[pl.ds(r0, STRIPE)], acc_out.at[c, pl.ds(r0, STRIPE)])


_RB = 1000


def _tc_head_body(acc_ref, xs_ref, dinv_ref, wc_ref, bc_ref, wl_ref, bl_ref, o_ref):
    acc = acc_ref[...]
    xsp = xs_ref[...]
    dinv = dinv_ref[...]
    agg = jnp.concatenate([acc[0] + xsp[0], acc[1] + xsp[1]], axis=1) * dinv
    h = jnp.dot(agg.astype(jnp.bfloat16), wc_ref[...].astype(jnp.bfloat16),
                preferred_element_type=jnp.float32) + bc_ref[...]
    h = jnp.maximum(h, 0.0)
    o_ref[...] = jnp.dot(h.astype(jnp.bfloat16), wl_ref[...].astype(jnp.bfloat16),
                         preferred_element_type=jnp.float32) + bl_ref[...]


def _tc_head(acc, xs_parts, dinv, W_conv, b_conv, W_lin, b_lin):
    nblk = N_NODES // _RB
    return pl.pallas_call(
        _tc_head_body,
        grid=(nblk,),
        in_specs=[
            pl.BlockSpec((2, _RB, HALF), lambda i: (0, i, 0)),
            pl.BlockSpec((2, _RB, HALF), lambda i: (0, i, 0)),
            pl.BlockSpec((_RB, 1), lambda i: (i, 0)),
            pl.BlockSpec((IN_DIM, HID_DIM), lambda i: (0, 0)),
            pl.BlockSpec((1, HID_DIM), lambda i: (0, 0)),
            pl.BlockSpec((HID_DIM, OUT_DIM), lambda i: (0, 0)),
            pl.BlockSpec((1, OUT_DIM), lambda i: (0, 0)),
        ],
        out_specs=pl.BlockSpec((_RB, OUT_DIM), lambda i: (i, 0)),
        out_shape=jax.ShapeDtypeStruct((N_NODES, OUT_DIM), jnp.float32),
    )(acc, xs_parts, dinv, W_conv, b_conv, W_lin, b_lin)


def kernel(x, edge_index, W_conv, b_conv, W_lin, b_lin):
    src = edge_index[0].astype(jnp.int32)
    dst = edge_index[1].astype(jnp.int32)
    pad = EP - N_EDGES
    srcp = jnp.concatenate([src, jnp.zeros((pad,), jnp.int32)])
    dstp = jnp.concatenate([dst, jnp.full((pad,), N_NODES, jnp.int32)])
    dst2d = dstp.reshape(NCHUNK, 128)
    src_all = jnp.stack([srcp, srcp + N_NODES]).reshape(2, NCHUNK, 128)

    ones128 = jnp.ones((128, 128), jnp.float32)
    zeros128 = jnp.zeros((NP, HALF), jnp.float32)

    hist = _sc_histogram(dst2d, zeros128, ones128)
    xs_parts, dinv = _tc_norm(hist, x)
    xs_cat = xs_parts.reshape(2 * N_NODES, HALF)
    acc = _sc_segsum(src_all, dst2d, xs_cat, zeros128)
    return _tc_head(acc, xs_parts, dinv, W_conv,
                    b_conv.reshape(1, HID_DIM), W_lin, b_lin.reshape(1, OUT_DIM))
